# baseline (speedup 1.0000x reference)
import functools

import jax
import jax.numpy as jnp
from jax import lax
from jax.experimental import pallas as pl
from jax.experimental.pallas import tpu as pltpu
from jax.experimental.pallas import tpu_sc as plsc

N = 10000
D = 128
E = 320000
SLOPE = 0.2

NC = 2
NS = 16
NW = NC * NS
CH = 64
TCH = E // CH
BASE_CNT = TCH // NW
EXTRA = TCH - BASE_CNT * NW
ZCH = 40
NZ = N // ZCH
ROWS_PER_WRITER = 1000


def _mm_body(h_ref, w_ref, a_ref, hw_ref, st_ref):
    hw = lax.dot_general(h_ref[...], w_ref[...],
                         dimension_numbers=(((1,), (1,)), ((), ())),
                         preferred_element_type=jnp.float32)
    hw_ref[...] = hw
    st_ref[...] = jnp.dot(hw, a_ref[...], preferred_element_type=jnp.float32)


def _matmul(h, W0, Apad):
    grid = (10,)
    blk = N // grid[0]
    return pl.pallas_call(
        _mm_body,
        grid=grid,
        in_specs=[
            pl.BlockSpec((blk, D), lambda i: (i, 0)),
            pl.BlockSpec((D, D), lambda i: (0, 0)),
            pl.BlockSpec((D, 8), lambda i: (0, 0)),
        ],
        out_specs=[
            pl.BlockSpec((blk, D), lambda i: (i, 0)),
            pl.BlockSpec((blk, 8), lambda i: (i, 0)),
        ],
        out_shape=[
            jax.ShapeDtypeStruct((N, D), jnp.float32),
            jax.ShapeDtypeStruct((N, 8), jnp.float32),
        ],
    )(h, W0, Apad)


def _edge_kernel(hw, s, t, src, dst):
    mesh = plsc.VectorSubcoreMesh(core_axis_name="c", subcore_axis_name="s")

    @functools.partial(
        pl.kernel,
        mesh=mesh,
        compiler_params=pltpu.CompilerParams(needs_layout_passes=False),
        out_type=[
            jax.ShapeDtypeStruct((NC, N, D), jnp.float32),
            jax.ShapeDtypeStruct((NC * N,), jnp.float32),
        ],
        scratch_types=[
            pltpu.VMEM((N,), jnp.float32),
            pltpu.VMEM((N,), jnp.float32),
            pltpu.VMEM((2, CH), jnp.int32),
            pltpu.VMEM((2, CH), jnp.int32),
            pltpu.VMEM((2, CH), jnp.int32),
            pltpu.VMEM((2, CH, D), jnp.float32),
            pltpu.VMEM((CH, D), jnp.float32),
            pltpu.VMEM((2, CH), jnp.float32),
            pltpu.VMEM((ROWS_PER_WRITER,), jnp.float32),
            pltpu.VMEM_SHARED((N, D), jnp.float32),
            pltpu.VMEM_SHARED((N,), jnp.float32),
            pltpu.SemaphoreType.DMA,
            pltpu.SemaphoreType.DMA,
            pltpu.SemaphoreType.DMA,
            pltpu.SemaphoreType.DMA,
            pltpu.SemaphoreType.DMA,
            pltpu.SemaphoreType.DMA,
            pltpu.SemaphoreType.DMA,
            pltpu.SemaphoreType.DMA,
        ],
    )
    def k(hw_hbm, s_hbm, t_hbm, src_hbm, dst_hbm, acc_out, den_out,
          sv, tv, sidx, didx, didx_s, rowsb, rowsf, ebuf, dbuf, acc_s, den_s,
          isem0, isem1, gsem0, gsem1, rsem0, rsem1, esem0, esem1):
        isem = (isem0, isem1)
        gsem = (gsem0, gsem1)
        rsem = (rsem0, rsem1)
        esem = (esem0, esem1)
        cid = lax.axis_index("c")
        sid = lax.axis_index("s")
        wid = sid * NC + cid

        pltpu.async_copy(s_hbm, sv, gsem0)
        pltpu.async_copy(t_hbm, tv, gsem1)

        def zrow(r, _):
            for q in range(D // 16):
                rowsf[r, pl.ds(q * 16, 16)] = jnp.zeros((16,), jnp.float32)
            return _
        lax.fori_loop(0, ZCH, zrow, None)
        for q in range((ZCH + 15) // 16):
            dbuf[pl.ds(q * 16, 16)] = jnp.zeros((16,), jnp.float32)

        zcnt = (NZ - sid + NS - 1) // NS

        def zchunk(kk, _):
            ch = sid + NS * kk
            pltpu.async_copy(rowsf.at[pl.ds(0, ZCH)],
                             acc_s.at[pl.ds(ch * ZCH, ZCH)], rsem0)
            pltpu.async_copy(dbuf.at[pl.ds(0, ZCH)],
                             den_s.at[pl.ds(ch * ZCH, ZCH)], rsem1)
            return _
        lax.fori_loop(0, zcnt, zchunk, None)

        def zwait(kk, _):
            pltpu.make_async_copy(rowsf.at[pl.ds(0, ZCH)],
                                  acc_s.at[pl.ds(0, ZCH)], rsem0).wait()
            pltpu.make_async_copy(dbuf.at[pl.ds(0, ZCH)],
                                  den_s.at[pl.ds(0, ZCH)], rsem1).wait()
            return _
        lax.fori_loop(0, zcnt, zwait, None)
        pltpu.make_async_copy(s_hbm, sv, gsem0).wait()
        pltpu.make_async_copy(t_hbm, tv, gsem1).wait()

        plsc.subcore_barrier()

        cnt = jnp.where(wid < EXTRA, BASE_CNT + 1, BASE_CNT)

        def idx_base(j):
            return (wid + NW * j) * CH

        def issue_idx(j, p):
            base = idx_base(j)
            pltpu.async_copy(src_hbm.at[pl.ds(base, CH)], sidx.at[p], isem[p])
            pltpu.async_copy(dst_hbm.at[pl.ds(base, CH)], didx.at[p], isem[p])

        def wait_idx(p):
            pltpu.make_async_copy(src_hbm.at[pl.ds(0, CH)], sidx.at[p],
                                  isem[p]).wait()
            pltpu.make_async_copy(dst_hbm.at[pl.ds(0, CH)], didx.at[p],
                                  isem[p]).wait()

        def issue_gather(p):
            pltpu.async_copy(hw_hbm.at[sidx.at[p]], rowsb.at[p], gsem[p])

        def wait_gather(p):
            pltpu.make_async_copy(hw_hbm.at[sidx.at[p]], rowsb.at[p],
                                  gsem[p]).wait()

        def wait_rowscat(p):
            pltpu.make_async_copy(rowsf, acc_s.at[didx_s.at[p]],
                                  rsem0).wait()

        def wait_escat(p):
            pltpu.make_async_copy(ebuf.at[p], den_s.at[didx_s.at[p]],
                                  esem[p]).wait()

        def step(j, p, q):
            @pl.when(j >= 2)
            def _():
                wait_escat(p)

            @pl.when(j + 1 < cnt)
            def _():
                wait_idx(q)
                issue_gather(q)

            for g in range(CH // 16):
                svi = sidx[p, pl.ds(g * 16, 16)]
                dvi = didx[p, pl.ds(g * 16, 16)]
                sg = plsc.load_gather(sv, [svi])
                tg = plsc.load_gather(tv, [dvi])
                x = sg + tg
                x = jnp.maximum(x, x * SLOPE)
                ex = jnp.exp(x)
                ex = jnp.minimum(jnp.maximum(ex, 0.005), 10.0)
                ebuf[p, pl.ds(g * 16, 16)] = ex
                didx_s[p, pl.ds(g * 16, 16)] = dvi

            pltpu.async_copy(ebuf.at[p], den_s.at[didx_s.at[p]], esem[p],
                             add=True)
            wait_gather(p)

            @pl.when(j >= 1)
            def _():
                wait_rowscat(q)

            def scale(r, _):
                es = plsc.load_gather(ebuf.at[p],
                                      [jnp.full((16,), r, jnp.int32)])
                for gg in range(D // 16):
                    rowsf[r, pl.ds(gg * 16, 16)] = (
                        rowsb[p, r, pl.ds(gg * 16, 16)] * es)
                return _
            lax.fori_loop(0, CH, scale, None)

            pltpu.async_copy(rowsf, acc_s.at[didx_s.at[p]], rsem0,
                             add=True)

            @pl.when(j + 2 < cnt)
            def _():
                issue_idx(j + 2, p)

        issue_idx(0, 0)
        issue_idx(1, 1)
        wait_idx(0)
        issue_gather(0)

        def pair(kk, _):
            j0 = 2 * kk
            step(j0, 0, 1)

            @pl.when(j0 + 1 < cnt)
            def _():
                step(j0 + 1, 1, 0)
            return _
        lax.fori_loop(0, (cnt + 1) // 2, pair, None)

        wait_rowscat(0)
        wait_escat(0)
        wait_escat(1)

        plsc.subcore_barrier()

        @pl.when(sid < N // ROWS_PER_WRITER)
        def _():
            r0 = sid * ROWS_PER_WRITER
            pltpu.sync_copy(acc_s.at[pl.ds(r0, ROWS_PER_WRITER)],
                            acc_out.at[cid, pl.ds(r0, ROWS_PER_WRITER)])
            pltpu.sync_copy(den_s.at[pl.ds(r0, ROWS_PER_WRITER)], dbuf)
            pltpu.sync_copy(dbuf,
                            den_out.at[pl.ds(cid * N + r0, ROWS_PER_WRITER)])

    return k(hw, s, t, src, dst)


def _merge_body(acc_ref, den_ref, out_ref):
    a = acc_ref[0] + acc_ref[1]
    d = den_ref[:, 0:1] + den_ref[:, 1:2]
    out_ref[...] = a / jnp.maximum(d, 1e-30)


def _merge(acc, den_t):
    grid = (10,)
    blk = N // grid[0]
    return pl.pallas_call(
        _merge_body,
        grid=grid,
        in_specs=[
            pl.BlockSpec((NC, blk, D), lambda i: (0, i, 0)),
            pl.BlockSpec((blk, NC), lambda i: (i, 0)),
        ],
        out_specs=pl.BlockSpec((blk, D), lambda i: (i, 0)),
        out_shape=jax.ShapeDtypeStruct((N, D), jnp.float32),
    )(acc, den_t)


def kernel(h, edges, W0, A0):
    a_pair = jnp.transpose(jnp.reshape(A0[0], (2, D)))
    Apad = jnp.concatenate([a_pair, jnp.zeros((D, 6), jnp.float32)], axis=1)
    hw, st = _matmul(h, W0, Apad)
    s = st[:, 0]
    t = st[:, 1]
    acc, den = _edge_kernel(hw, s, t, edges[0], edges[1])
    den_t = jnp.transpose(jnp.reshape(den, (NC, N)))
    out = _merge(acc, den_t)
    return out


# speedup vs baseline: 2.2229x; 2.2229x over previous
import functools

import jax
import jax.numpy as jnp
from jax import lax
from jax.experimental import pallas as pl
from jax.experimental.pallas import tpu as pltpu
from jax.experimental.pallas import tpu_sc as plsc

N = 10000
D = 128
E = 320000
SLOPE = 0.2

NC = 2
NS = 16
NW = NC * NS
CH = 80
TCH = E // CH
BASE_CNT = TCH // NW
EXTRA = TCH - BASE_CNT * NW
ZCH = 40
NZ = N // ZCH
ROWS_PER_WRITER = 1000


def _mm_body(h_ref, w_ref, a_ref, hw_ref, st_ref):
    hw = lax.dot_general(h_ref[...], w_ref[...],
                         dimension_numbers=(((1,), (1,)), ((), ())),
                         preferred_element_type=jnp.float32)
    hw_ref[...] = hw
    st_ref[...] = jnp.dot(hw, a_ref[...], preferred_element_type=jnp.float32)


def _matmul(h, W0, Apad):
    grid = (10,)
    blk = N // grid[0]
    return pl.pallas_call(
        _mm_body,
        grid=grid,
        in_specs=[
            pl.BlockSpec((blk, D), lambda i: (i, 0)),
            pl.BlockSpec((D, D), lambda i: (0, 0)),
            pl.BlockSpec((D, 8), lambda i: (0, 0)),
        ],
        out_specs=[
            pl.BlockSpec((blk, D), lambda i: (i, 0)),
            pl.BlockSpec((blk, 8), lambda i: (i, 0)),
        ],
        out_shape=[
            jax.ShapeDtypeStruct((N, D), jnp.float32),
            jax.ShapeDtypeStruct((N, 8), jnp.float32),
        ],
    )(h, W0, Apad)


def _edge_kernel(hw, s, t, src, dst):
    mesh = plsc.VectorSubcoreMesh(core_axis_name="c", subcore_axis_name="s")

    @functools.partial(
        pl.kernel,
        mesh=mesh,
        compiler_params=pltpu.CompilerParams(needs_layout_passes=False),
        out_type=[
            jax.ShapeDtypeStruct((NC, N, D), jnp.float32),
            jax.ShapeDtypeStruct((NC * N,), jnp.float32),
        ],
        scratch_types=[
            pltpu.VMEM((N,), jnp.float32),
            pltpu.VMEM((N,), jnp.float32),
            pltpu.VMEM((2, CH), jnp.int32),
            pltpu.VMEM((2, CH), jnp.int32),
            pltpu.VMEM((2, CH), jnp.int32),
            pltpu.VMEM((2, CH, D), jnp.float32),
            pltpu.VMEM((2, CH), jnp.float32),
            pltpu.VMEM((ZCH, D), jnp.float32),
            pltpu.VMEM((ROWS_PER_WRITER,), jnp.float32),
            pltpu.VMEM_SHARED((N, D), jnp.float32),
            pltpu.VMEM_SHARED((N,), jnp.float32),
            pltpu.SemaphoreType.DMA,
            pltpu.SemaphoreType.DMA,
            pltpu.SemaphoreType.DMA,
            pltpu.SemaphoreType.DMA,
            pltpu.SemaphoreType.DMA,
            pltpu.SemaphoreType.DMA,
            pltpu.SemaphoreType.DMA,
            pltpu.SemaphoreType.DMA,
        ],
    )
    def k(hw_hbm, s_hbm, t_hbm, src_hbm, dst_hbm, acc_out, den_out,
          sv, tv, sidx, didx, didx_s, rows, ebuf, zbuf, dbuf, acc_s, den_s,
          isem0, isem1, gsem0, gsem1, rsem0, rsem1, esem0, esem1):
        isem = (isem0, isem1)
        gsem = (gsem0, gsem1)
        rsem = (rsem0, rsem1)
        esem = (esem0, esem1)
        cid = lax.axis_index("c")
        sid = lax.axis_index("s")
        wid = sid * NC + cid

        pltpu.async_copy(s_hbm, sv, gsem0)
        pltpu.async_copy(t_hbm, tv, gsem1)

        def zrow(r, _):
            for q in range(D // 16):
                zbuf[r, pl.ds(q * 16, 16)] = jnp.zeros((16,), jnp.float32)
            return _
        lax.fori_loop(0, ZCH, zrow, None)

        zcnt = (NZ - sid + NS - 1) // NS

        def zchunk(kk, _):
            ch = sid + NS * kk
            pltpu.async_copy(zbuf, acc_s.at[pl.ds(ch * ZCH, ZCH)], rsem0)
            pltpu.async_copy(zbuf.at[0, pl.ds(0, ZCH)],
                            den_s.at[pl.ds(ch * ZCH, ZCH)], rsem1)
            return _
        lax.fori_loop(0, zcnt, zchunk, None)

        def zwait(kk, _):
            pltpu.make_async_copy(zbuf, acc_s.at[pl.ds(0, ZCH)],
                                  rsem0).wait()
            pltpu.make_async_copy(zbuf.at[0, pl.ds(0, ZCH)],
                                  den_s.at[pl.ds(0, ZCH)], rsem1).wait()
            return _
        lax.fori_loop(0, zcnt, zwait, None)
        pltpu.make_async_copy(s_hbm, sv, gsem0).wait()
        pltpu.make_async_copy(t_hbm, tv, gsem1).wait()

        plsc.subcore_barrier()

        cnt = jnp.where(wid < EXTRA, BASE_CNT + 1, BASE_CNT)

        def idx_base(j):
            return (wid + NW * j) * CH

        def issue_idx(j, p):
            base = idx_base(j)
            pltpu.async_copy(src_hbm.at[pl.ds(base, CH)], sidx.at[p], isem[p])
            pltpu.async_copy(dst_hbm.at[pl.ds(base, CH)], didx.at[p], isem[p])

        def wait_idx(p):
            pltpu.make_async_copy(src_hbm.at[pl.ds(0, CH)], sidx.at[p],
                                  isem[p]).wait()
            pltpu.make_async_copy(dst_hbm.at[pl.ds(0, CH)], didx.at[p],
                                  isem[p]).wait()

        def issue_gather(p):
            pltpu.async_copy(hw_hbm.at[sidx.at[p]], rows.at[p], gsem[p])

        def wait_gather(p):
            pltpu.make_async_copy(hw_hbm.at[sidx.at[p]], rows.at[p],
                                  gsem[p]).wait()

        def wait_rowscat(p):
            pltpu.make_async_copy(rows.at[p], acc_s.at[didx_s.at[p]],
                                  rsem[p]).wait()

        def wait_escat(p):
            pltpu.make_async_copy(ebuf.at[p], den_s.at[didx_s.at[p]],
                                  esem[p]).wait()

        def step(j, p, q):
            @pl.when(j >= 2)
            def _():
                wait_escat(p)

            for g in range(CH // 16):
                svi = sidx[p, pl.ds(g * 16, 16)]
                dvi = didx[p, pl.ds(g * 16, 16)]
                sg = plsc.load_gather(sv, [svi])
                tg = plsc.load_gather(tv, [dvi])
                x = sg + tg
                x = jnp.maximum(x, x * SLOPE)
                ex = jnp.exp(x)
                ex = jnp.minimum(jnp.maximum(ex, 0.005), 10.0)
                ebuf[p, pl.ds(g * 16, 16)] = ex
                didx_s[p, pl.ds(g * 16, 16)] = dvi

            pltpu.async_copy(ebuf.at[p], den_s.at[didx_s.at[p]], esem[p],
                             add=True)

            @pl.when(j + 1 < cnt)
            def _():
                wait_idx(q)

                @pl.when(j >= 1)
                def _():
                    wait_rowscat(q)
                issue_gather(q)

            wait_gather(p)

            def scale(r, _):
                es = plsc.load_gather(ebuf.at[p],
                                      [jnp.full((16,), r, jnp.int32)])
                for gg in range(D // 16):
                    rows[p, r, pl.ds(gg * 16, 16)] = (
                        rows[p, r, pl.ds(gg * 16, 16)] * es)
                return _
            lax.fori_loop(0, CH, scale, None)

            pltpu.async_copy(rows.at[p], acc_s.at[didx_s.at[p]], rsem[p],
                             add=True)

            @pl.when(j + 2 < cnt)
            def _():
                issue_idx(j + 2, p)

        issue_idx(0, 0)
        issue_idx(1, 1)
        wait_idx(0)
        issue_gather(0)

        def pair(kk, _):
            j0 = 2 * kk
            step(j0, 0, 1)

            @pl.when(j0 + 1 < cnt)
            def _():
                step(j0 + 1, 1, 0)
            return _
        lax.fori_loop(0, (cnt + 1) // 2, pair, None)

        wait_rowscat(0)
        wait_rowscat(1)
        wait_escat(0)
        wait_escat(1)

        plsc.subcore_barrier()

        @pl.when(sid < N // ROWS_PER_WRITER)
        def _():
            r0 = sid * ROWS_PER_WRITER
            pltpu.sync_copy(acc_s.at[pl.ds(r0, ROWS_PER_WRITER)],
                            acc_out.at[cid, pl.ds(r0, ROWS_PER_WRITER)])
            pltpu.sync_copy(den_s.at[pl.ds(r0, ROWS_PER_WRITER)], dbuf)
            pltpu.sync_copy(dbuf,
                            den_out.at[pl.ds(cid * N + r0, ROWS_PER_WRITER)])

    return k(hw, s, t, src, dst)


def _merge_body(acc_ref, den_ref, out_ref):
    a = acc_ref[0] + acc_ref[1]
    d = den_ref[:, 0:1] + den_ref[:, 1:2]
    out_ref[...] = a / jnp.maximum(d, 1e-30)


def _merge(acc, den_t):
    grid = (10,)
    blk = N // grid[0]
    return pl.pallas_call(
        _merge_body,
        grid=grid,
        in_specs=[
            pl.BlockSpec((NC, blk, D), lambda i: (0, i, 0)),
            pl.BlockSpec((blk, NC), lambda i: (i, 0)),
        ],
        out_specs=pl.BlockSpec((blk, D), lambda i: (i, 0)),
        out_shape=jax.ShapeDtypeStruct((N, D), jnp.float32),
    )(acc, den_t)


def kernel(h, edges, W0, A0):
    a_pair = jnp.transpose(jnp.reshape(A0[0], (2, D)))
    Apad = jnp.concatenate([a_pair, jnp.zeros((D, 6), jnp.float32)], axis=1)
    hw, st = _matmul(h, W0, Apad)
    s = st[:, 0]
    t = st[:, 1]
    acc, den = _edge_kernel(hw, s, t, edges[0], edges[1])
    den_t = jnp.transpose(jnp.reshape(den, (NC, N)))
    out = _merge(acc, den_t)
    return out


# speedup vs baseline: 2.6557x; 1.1947x over previous
import functools

import jax
import jax.numpy as jnp
from jax import lax
from jax.experimental import pallas as pl
from jax.experimental.pallas import tpu as pltpu
from jax.experimental.pallas import tpu_sc as plsc

N = 10000
D = 128
E = 320000
SLOPE = 0.2

NC = 2
NS = 16
NW = NC * NS
CH = 80
TCH = E // CH
BASE_CNT = TCH // NW
EXTRA = TCH - BASE_CNT * NW
ZCH = 40
NZ = N // ZCH
ROWS_PER_WRITER = 1000


def _mm_body(h_ref, w_ref, a_ref, hw_ref, st_ref):
    hw = lax.dot_general(h_ref[...], w_ref[...],
                         dimension_numbers=(((1,), (1,)), ((), ())),
                         preferred_element_type=jnp.float32)
    hw_ref[...] = hw
    st_ref[...] = jnp.dot(hw, a_ref[...], preferred_element_type=jnp.float32)


def _matmul(h, W0, Apad):
    grid = (10,)
    blk = N // grid[0]
    return pl.pallas_call(
        _mm_body,
        grid=grid,
        in_specs=[
            pl.BlockSpec((blk, D), lambda i: (i, 0)),
            pl.BlockSpec((D, D), lambda i: (0, 0)),
            pl.BlockSpec((D, 8), lambda i: (0, 0)),
        ],
        out_specs=[
            pl.BlockSpec((blk, D), lambda i: (i, 0)),
            pl.BlockSpec((blk, 8), lambda i: (i, 0)),
        ],
        out_shape=[
            jax.ShapeDtypeStruct((N, D), jnp.float32),
            jax.ShapeDtypeStruct((N, 8), jnp.float32),
        ],
    )(h, W0, Apad)


def _edge_kernel(hw, stv, src, dst):
    mesh = plsc.VectorSubcoreMesh(core_axis_name="c", subcore_axis_name="s")

    @functools.partial(
        pl.kernel,
        mesh=mesh,
        compiler_params=pltpu.CompilerParams(needs_layout_passes=False),
        out_type=[
            jax.ShapeDtypeStruct((NC, N, D), jnp.float32),
            jax.ShapeDtypeStruct((NC * N,), jnp.float32),
        ],
        scratch_types=[
            pltpu.VMEM((N,), jnp.int32),
            pltpu.VMEM((3, CH), jnp.int32),
            pltpu.VMEM((3, CH), jnp.int32),
            pltpu.VMEM((3, CH), jnp.int32),
            pltpu.VMEM((3, CH, D), jnp.float32),
            pltpu.VMEM((3, CH), jnp.float32),
            pltpu.VMEM((ZCH, D), jnp.float32),
            pltpu.VMEM((ROWS_PER_WRITER,), jnp.float32),
            pltpu.VMEM_SHARED((N, D), jnp.float32),
            pltpu.VMEM_SHARED((N,), jnp.float32),
            pltpu.SemaphoreType.DMA,
            pltpu.SemaphoreType.DMA,
            pltpu.SemaphoreType.DMA,
            pltpu.SemaphoreType.DMA,
            pltpu.SemaphoreType.DMA,
            pltpu.SemaphoreType.DMA,
            pltpu.SemaphoreType.DMA,
            pltpu.SemaphoreType.DMA,
            pltpu.SemaphoreType.DMA,
            pltpu.SemaphoreType.DMA,
            pltpu.SemaphoreType.DMA,
            pltpu.SemaphoreType.DMA,
        ],
    )
    def k(hw_hbm, stv_hbm, src_hbm, dst_hbm, acc_out, den_out,
          stv, sidx, didx, didx_s, rows, ebuf, zbuf, dbuf, acc_s, den_s,
          isem0, isem1, isem2, gsem0, gsem1, gsem2,
          rsem0, rsem1, rsem2, esem0, esem1, esem2):
        isem = (isem0, isem1, isem2)
        gsem = (gsem0, gsem1, gsem2)
        rsem = (rsem0, rsem1, rsem2)
        esem = (esem0, esem1, esem2)
        cid = lax.axis_index("c")
        sid = lax.axis_index("s")
        wid = sid * NC + cid

        pltpu.async_copy(stv_hbm, stv, gsem0)

        def zrow(r, _):
            for q in range(D // 16):
                zbuf[r, pl.ds(q * 16, 16)] = jnp.zeros((16,), jnp.float32)
            return _
        lax.fori_loop(0, ZCH, zrow, None)

        zcnt = (NZ - sid + NS - 1) // NS

        def zchunk(kk, _):
            ch = sid + NS * kk
            pltpu.async_copy(zbuf, acc_s.at[pl.ds(ch * ZCH, ZCH)], rsem0)
            pltpu.async_copy(zbuf.at[0, pl.ds(0, ZCH)],
                            den_s.at[pl.ds(ch * ZCH, ZCH)], rsem1)
            return _
        lax.fori_loop(0, zcnt, zchunk, None)

        def zwait(kk, _):
            pltpu.make_async_copy(zbuf, acc_s.at[pl.ds(0, ZCH)],
                                  rsem0).wait()
            pltpu.make_async_copy(zbuf.at[0, pl.ds(0, ZCH)],
                                  den_s.at[pl.ds(0, ZCH)], rsem1).wait()
            return _
        lax.fori_loop(0, zcnt, zwait, None)
        pltpu.make_async_copy(stv_hbm, stv, gsem0).wait()

        plsc.subcore_barrier()

        cnt = jnp.where(wid < EXTRA, BASE_CNT + 1, BASE_CNT)

        def idx_base(j):
            return (wid + NW * j) * CH

        def issue_idx(j, p):
            base = idx_base(j)
            pltpu.async_copy(src_hbm.at[pl.ds(base, CH)], sidx.at[p], isem[p])
            pltpu.async_copy(dst_hbm.at[pl.ds(base, CH)], didx.at[p], isem[p])

        def wait_idx(p):
            pltpu.make_async_copy(src_hbm.at[pl.ds(0, CH)], sidx.at[p],
                                  isem[p]).wait()
            pltpu.make_async_copy(dst_hbm.at[pl.ds(0, CH)], didx.at[p],
                                  isem[p]).wait()

        def issue_gather(p):
            pltpu.async_copy(hw_hbm.at[sidx.at[p]], rows.at[p], gsem[p])

        def wait_gather(p):
            pltpu.make_async_copy(hw_hbm.at[sidx.at[p]], rows.at[p],
                                  gsem[p]).wait()

        def wait_rowscat(p):
            pltpu.make_async_copy(rows.at[p], acc_s.at[didx_s.at[p]],
                                  rsem[p]).wait()

        def wait_escat(p):
            pltpu.make_async_copy(ebuf.at[p], den_s.at[didx_s.at[p]],
                                  esem[p]).wait()

        def step(j, p, pn):
            @pl.when(j >= 3)
            def _():
                wait_escat(p)

            @pl.when(j + 1 < cnt)
            def _():
                wait_idx(pn)

                @pl.when(j >= 2)
                def _():
                    wait_rowscat(pn)
                issue_gather(pn)

            for g in range(CH // 16):
                svi = sidx[p, pl.ds(g * 16, 16)]
                dvi = didx[p, pl.ds(g * 16, 16)]
                ps = plsc.load_gather(stv, [svi])
                pd = plsc.load_gather(stv, [dvi])
                sg, _unused = plsc.unpack(plsc.bitcast(ps, jnp.bfloat16),
                                          format=plsc.PackFormat.INTERLEAVED)
                _unused2, tg = plsc.unpack(plsc.bitcast(pd, jnp.bfloat16),
                                           format=plsc.PackFormat.INTERLEAVED)
                x = sg + tg
                x = jnp.maximum(x, x * SLOPE)
                ex = jnp.exp(x)
                ex = jnp.minimum(jnp.maximum(ex, 0.005), 10.0)
                ebuf[p, pl.ds(g * 16, 16)] = ex
                didx_s[p, pl.ds(g * 16, 16)] = dvi

            pltpu.async_copy(ebuf.at[p], den_s.at[didx_s.at[p]], esem[p],
                             add=True)
            wait_gather(p)

            def scale(r, _):
                es = plsc.load_gather(ebuf.at[p],
                                      [jnp.full((16,), r, jnp.int32)])
                for gg in range(D // 16):
                    rows[p, r, pl.ds(gg * 16, 16)] = (
                        rows[p, r, pl.ds(gg * 16, 16)] * es)
                return _
            lax.fori_loop(0, CH, scale, None)

            pltpu.async_copy(rows.at[p], acc_s.at[didx_s.at[p]], rsem[p],
                             add=True)

            @pl.when(j + 3 < cnt)
            def _():
                issue_idx(j + 3, p)

        issue_idx(0, 0)
        issue_idx(1, 1)
        issue_idx(2, 2)
        wait_idx(0)
        issue_gather(0)

        def triple(kk, _):
            j0 = 3 * kk
            step(j0, 0, 1)

            @pl.when(j0 + 1 < cnt)
            def _():
                step(j0 + 1, 1, 2)

            @pl.when(j0 + 2 < cnt)
            def _():
                step(j0 + 2, 2, 0)
            return _
        lax.fori_loop(0, (cnt + 2) // 3, triple, None)

        wait_rowscat(0)
        wait_rowscat(1)
        wait_rowscat(2)
        wait_escat(0)
        wait_escat(1)
        wait_escat(2)

        plsc.subcore_barrier()

        @pl.when(sid < N // ROWS_PER_WRITER)
        def _():
            r0 = sid * ROWS_PER_WRITER
            pltpu.sync_copy(acc_s.at[pl.ds(r0, ROWS_PER_WRITER)],
                            acc_out.at[cid, pl.ds(r0, ROWS_PER_WRITER)])
            pltpu.sync_copy(den_s.at[pl.ds(r0, ROWS_PER_WRITER)], dbuf)
            pltpu.sync_copy(dbuf,
                            den_out.at[pl.ds(cid * N + r0, ROWS_PER_WRITER)])

    return k(hw, stv, src, dst)


def _merge_body(acc_ref, den_ref, out_ref):
    a = acc_ref[0] + acc_ref[1]
    d = den_ref[:, 0:1] + den_ref[:, 1:2]
    out_ref[...] = a / jnp.maximum(d, 1e-30)


def _merge(acc, den_t):
    grid = (10,)
    blk = N // grid[0]
    return pl.pallas_call(
        _merge_body,
        grid=grid,
        in_specs=[
            pl.BlockSpec((NC, blk, D), lambda i: (0, i, 0)),
            pl.BlockSpec((blk, NC), lambda i: (i, 0)),
        ],
        out_specs=pl.BlockSpec((blk, D), lambda i: (i, 0)),
        out_shape=jax.ShapeDtypeStruct((N, D), jnp.float32),
    )(acc, den_t)


def kernel(h, edges, W0, A0):
    a_pair = jnp.transpose(jnp.reshape(A0[0], (2, D)))
    Apad = jnp.concatenate([a_pair, jnp.zeros((D, 6), jnp.float32)], axis=1)
    hw, st = _matmul(h, W0, Apad)
    st_bf = st[:, :2].astype(jnp.bfloat16)
    stv = lax.bitcast_convert_type(st_bf, jnp.int32)
    acc, den = _edge_kernel(hw, stv, edges[0], edges[1])
    den_t = jnp.transpose(jnp.reshape(den, (NC, N)))
    out = _merge(acc, den_t)
    return out
